# baseline (speedup 1.0000x reference)
import functools

import jax
import jax.numpy as jnp
from jax import lax
from jax.experimental import pallas as pl
from jax.experimental.pallas import tpu as pltpu
from jax.experimental.pallas import tpu_sc as plsc

_BATCH = 128
_VOCAB = 100000
_NC = 2
_NS = 16
_NW = _NC * _NS
_RPW = _BATCH // _NW
_CHUNK = 10000
_CPR = _VOCAB // _CHUNK
_NCHUNKS = _RPW * _CPR
_LANES = 16
_NCHAIN = 5


def _lane_gather(x, idx):
    return lax.gather(
        x,
        idx[:, None],
        dimension_numbers=lax.GatherDimensionNumbers(
            offset_dims=(), collapsed_slice_dims=(0,), start_index_map=(0,)),
        slice_sizes=(1,),
        mode=lax.GatherScatterMode.PROMISE_IN_BOUNDS,
    )


def _sc_argmax_body(x_hbm, out_hbm, buf0, buf1, buf2, res_v, sem0, sem1,
                    sem2):
    wid = lax.axis_index("s") * _NC + lax.axis_index("c")
    row0 = wid * _RPW
    bufs = (buf0, buf1)
    sems = (sem0, sem1)

    base = row0 * _VOCAB

    def start(g, b):
        pltpu.make_async_copy(
            x_hbm.at[pl.ds(base + g * _CHUNK, _CHUNK)],
            bufs[b],
            sems[b],
        ).start()

    start(0, 0)
    start(1, 1)

    lane = lax.iota(jnp.int32, _LANES)
    res = jnp.zeros((_LANES,), jnp.int32)
    neg_inf = jnp.full((_LANES,), -jnp.inf, jnp.float32)
    zeros = jnp.zeros((_LANES,), jnp.int32)

    for r in range(_RPW):
        @pl.loop(0, _CPR, init_carry=neg_inf, step=2)
        def chunk_loop(c, cm):
            for b in range(2):
                g = r * _CPR + c + b
                pltpu.make_async_copy(
                    x_hbm.at[pl.ds(0, _CHUNK)], bufs[b], sems[b]
                ).wait()

                @pl.loop(0, _CHUNK, init_carry=(neg_inf,) * _NCHAIN,
                         step=_LANES * _NCHAIN)
                def inner(off, ic):
                    return tuple(
                        jnp.maximum(ic[k],
                                    bufs[b][pl.ds(off + k * _LANES, _LANES)])
                        for k in range(_NCHAIN))

                @pl.when(g + 2 < _NCHUNKS)
                def _():
                    start(g + 2, b)

                m = inner[0]
                for k in range(1, _NCHAIN):
                    m = jnp.maximum(m, inner[k])
                for shift in (8, 4, 2, 1):
                    m = jnp.maximum(m, _lane_gather(m, lane ^ shift))
                cm = jnp.where(lane == c + b, m, cm)
            return cm

        cm = chunk_loop

        ci = lane
        for shift in (8, 4, 2, 1):
            ov = _lane_gather(cm, lane ^ shift)
            oi = _lane_gather(ci, lane ^ shift)
            p = (ov > cm) | ((ov == cm) & (oi < ci))
            cm = jnp.where(p, ov, cm)
            ci = jnp.where(p, oi, ci)

        c_star = ci[0]

        pltpu.make_async_copy(
            x_hbm.at[pl.ds(base + r * _VOCAB + c_star * _CHUNK, _CHUNK)],
            buf2,
            sem2,
        ).start()
        pltpu.make_async_copy(
            x_hbm.at[pl.ds(0, _CHUNK)], buf2, sem2
        ).wait()

        @pl.loop(0, _CHUNK,
                 init_carry=tuple((neg_inf, zeros) for _ in range(_NCHAIN)),
                 step=_LANES * _NCHAIN)
        def scan2(off, ic):
            basev = jnp.full((_LANES,), off, jnp.int32)
            nxt = []
            for k in range(_NCHAIN):
                m, mo = ic[k]
                v = buf2[pl.ds(off + k * _LANES, _LANES)]
                p = v > m
                nxt.append((
                    jnp.where(p, v, m),
                    jnp.where(p, basev, mo),
                ))
            return tuple(nxt)

        m, mi = scan2[0]
        mi = mi + lane
        for k in range(1, _NCHAIN):
            bm, bmi = scan2[k]
            bmi = bmi + (lane + k * _LANES)
            p = (bm > m) | ((bm == m) & (bmi < mi))
            m = jnp.where(p, bm, m)
            mi = jnp.where(p, bmi, mi)
        for shift in (8, 4, 2, 1):
            ov = _lane_gather(m, lane ^ shift)
            oi = _lane_gather(mi, lane ^ shift)
            p = (ov > m) | ((ov == m) & (oi < mi))
            m = jnp.where(p, ov, m)
            mi = jnp.where(p, oi, mi)

        res = jnp.where(lane == r, ci * _CHUNK + mi, res)

    res_v[...] = res
    pltpu.sync_copy(res_v, out_hbm.at[wid])


@jax.jit
def _sc_argmax(x):
    mesh = plsc.VectorSubcoreMesh(
        core_axis_name="c", subcore_axis_name="s",
        num_cores=_NC, num_subcores=_NS)
    f = pl.kernel(
        _sc_argmax_body,
        out_type=jax.ShapeDtypeStruct((_NW, _LANES), jnp.int32),
        mesh=mesh,
        scratch_types=[
            pltpu.VMEM((_CHUNK,), jnp.float32),
            pltpu.VMEM((_CHUNK,), jnp.float32),
            pltpu.VMEM((_CHUNK,), jnp.float32),
            pltpu.VMEM((_LANES,), jnp.int32),
            pltpu.SemaphoreType.DMA,
            pltpu.SemaphoreType.DMA,
            pltpu.SemaphoreType.DMA,
        ],
    )
    return f(x)


def kernel(logits):
    assert logits.shape == (_BATCH, _VOCAB)
    padded = _sc_argmax(logits.reshape(-1))
    return padded[:, :_RPW].reshape(_BATCH)


# speedup vs baseline: 1.0546x; 1.0546x over previous
import functools

import jax
import jax.numpy as jnp
from jax import lax
from jax.experimental import pallas as pl
from jax.experimental.pallas import tpu as pltpu
from jax.experimental.pallas import tpu_sc as plsc

_BATCH = 128
_VOCAB = 100000
_NC = 2
_NS = 16
_NW = _NC * _NS
_RPW = _BATCH // _NW
_CHUNK = 10000
_CPR = _VOCAB // _CHUNK
_NCHUNKS = _RPW * _CPR
_LANES = 16
_NCHAIN = 5


def _lane_gather(x, idx):
    return lax.gather(
        x,
        idx[:, None],
        dimension_numbers=lax.GatherDimensionNumbers(
            offset_dims=(), collapsed_slice_dims=(0,), start_index_map=(0,)),
        slice_sizes=(1,),
        mode=lax.GatherScatterMode.PROMISE_IN_BOUNDS,
    )


def _sc_argmax_body(x_hbm, out_hbm, buf0, buf1, buf2, buf3, buf4, res_v,
                    sem0, sem1, sem2, sem3, sem4):
    wid = lax.axis_index("s") * _NC + lax.axis_index("c")
    row0 = wid * _RPW
    bufs = (buf0, buf1, buf2, buf3)
    sems = (sem0, sem1, sem2, sem3)

    base = row0 * _VOCAB

    def start(g, b):
        pltpu.make_async_copy(
            x_hbm.at[pl.ds(base + g * _CHUNK, _CHUNK)],
            bufs[b],
            sems[b],
        ).start()

    for b in range(4):
        start(b, b)

    lane = lax.iota(jnp.int32, _LANES)
    res = jnp.zeros((_LANES,), jnp.int32)
    neg_inf = jnp.full((_LANES,), -jnp.inf, jnp.float32)
    zeros = jnp.zeros((_LANES,), jnp.int32)

    for r in range(_RPW):
        cm = neg_inf
        for c in range(_CPR):
            g = r * _CPR + c
            b = g % 4
            pltpu.make_async_copy(
                x_hbm.at[pl.ds(0, _CHUNK)], bufs[b], sems[b]
            ).wait()

            @pl.loop(0, _CHUNK, init_carry=(neg_inf,) * _NCHAIN,
                     step=_LANES * _NCHAIN)
            def inner(off, ic):
                return tuple(
                    jnp.maximum(ic[k],
                                bufs[b][pl.ds(off + k * _LANES, _LANES)])
                    for k in range(_NCHAIN))

            if g + 4 < _NCHUNKS:
                start(g + 4, b)

            m = inner[0]
            for k in range(1, _NCHAIN):
                m = jnp.maximum(m, inner[k])
            for shift in (8, 4, 2, 1):
                m = jnp.maximum(m, _lane_gather(m, lane ^ shift))
            cm = jnp.where(lane == c, m, cm)

        ci = lane
        for shift in (8, 4, 2, 1):
            ov = _lane_gather(cm, lane ^ shift)
            oi = _lane_gather(ci, lane ^ shift)
            p = (ov > cm) | ((ov == cm) & (oi < ci))
            cm = jnp.where(p, ov, cm)
            ci = jnp.where(p, oi, ci)

        c_star = ci[0]

        pltpu.make_async_copy(
            x_hbm.at[pl.ds(base + r * _VOCAB + c_star * _CHUNK, _CHUNK)],
            buf4,
            sem4,
        ).start()
        pltpu.make_async_copy(
            x_hbm.at[pl.ds(0, _CHUNK)], buf4, sem4
        ).wait()

        @pl.loop(0, _CHUNK,
                 init_carry=tuple((neg_inf, zeros) for _ in range(_NCHAIN)),
                 step=_LANES * _NCHAIN)
        def scan2(off, ic):
            basev = jnp.full((_LANES,), off, jnp.int32)
            nxt = []
            for k in range(_NCHAIN):
                m, mo = ic[k]
                v = buf4[pl.ds(off + k * _LANES, _LANES)]
                p = v > m
                nxt.append((
                    jnp.where(p, v, m),
                    jnp.where(p, basev, mo),
                ))
            return tuple(nxt)

        m, mi = scan2[0]
        mi = mi + lane
        for k in range(1, _NCHAIN):
            bm, bmi = scan2[k]
            bmi = bmi + (lane + k * _LANES)
            p = (bm > m) | ((bm == m) & (bmi < mi))
            m = jnp.where(p, bm, m)
            mi = jnp.where(p, bmi, mi)
        for shift in (8, 4, 2, 1):
            ov = _lane_gather(m, lane ^ shift)
            oi = _lane_gather(mi, lane ^ shift)
            p = (ov > m) | ((ov == m) & (oi < mi))
            m = jnp.where(p, ov, m)
            mi = jnp.where(p, oi, mi)

        res = jnp.where(lane == r, ci * _CHUNK + mi, res)

    res_v[...] = res
    pltpu.sync_copy(res_v, out_hbm.at[wid])


@jax.jit
def _sc_argmax(x):
    mesh = plsc.VectorSubcoreMesh(
        core_axis_name="c", subcore_axis_name="s",
        num_cores=_NC, num_subcores=_NS)
    f = pl.kernel(
        _sc_argmax_body,
        out_type=jax.ShapeDtypeStruct((_NW, _LANES), jnp.int32),
        mesh=mesh,
        scratch_types=[
            pltpu.VMEM((_CHUNK,), jnp.float32),
            pltpu.VMEM((_CHUNK,), jnp.float32),
            pltpu.VMEM((_CHUNK,), jnp.float32),
            pltpu.VMEM((_CHUNK,), jnp.float32),
            pltpu.VMEM((_CHUNK,), jnp.float32),
            pltpu.VMEM((_LANES,), jnp.int32),
            pltpu.SemaphoreType.DMA,
            pltpu.SemaphoreType.DMA,
            pltpu.SemaphoreType.DMA,
            pltpu.SemaphoreType.DMA,
            pltpu.SemaphoreType.DMA,
        ],
    )
    return f(x)


def kernel(logits):
    assert logits.shape == (_BATCH, _VOCAB)
    padded = _sc_argmax(logits.reshape(-1))
    return padded[:, :_RPW].reshape(_BATCH)
